# baseline (speedup 1.0000x reference)
import functools

import jax
import jax.numpy as jnp
from jax import lax
from jax.experimental import pallas as pl
from jax.experimental.pallas import tpu as pltpu
from jax.experimental.pallas import tpu_sc as plsc

N_NODES = 10000
N_EDGES = 320000
IN_FEAT = 128
OUT_FEAT = 128
NUM_RELS = 16
R17 = NUM_RELS + 1
HALF = OUT_FEAT // 2
NP = 2

NC = 2
NS = 16
NW = NC * NS
EPW = N_EDGES // NW
CHUNK = 500
NCH = 20
NBUF = 2
ACC_ROWS = N_NODES + 16
ZROWS = 80
NZB = N_NODES // ZROWS

NB = 10
BLK = N_NODES // NB


def _xw_body(f_ref, w_ref, o_ref):
    o_ref[...] = jnp.dot(f_ref[...], w_ref[0],
                         preferred_element_type=jnp.float32)


def _compute_xw(feat, w_all):
    return pl.pallas_call(
        _xw_body,
        grid=(NB, R17),
        in_specs=[
            pl.BlockSpec((BLK, IN_FEAT), lambda nb, r: (nb, 0)),
            pl.BlockSpec((1, IN_FEAT, OUT_FEAT), lambda nb, r: (r, 0, 0)),
        ],
        out_specs=pl.BlockSpec((BLK, OUT_FEAT), lambda nb, r: (r * NB + nb, 0)),
        out_shape=jax.ShapeDtypeStruct((R17 * N_NODES, OUT_FEAT), jnp.float32),
    )(feat, w_all)


def _sc_body(xw_hbm, g_hbm, d_hbm, out_hbm, gv, dv, bufs, zb, acc, sems):
    c = lax.axis_index("c")
    s = lax.axis_index("s")
    wid = c * NS + s

    pltpu.sync_copy(d_hbm.at[wid], dv)

    zvec = jnp.zeros((16,), jnp.float32)

    def _zfill(i, carry):
        zb[i // (HALF // 16), pl.ds((i % (HALF // 16)) * 16, 16)] = zvec
        return carry

    lax.fori_loop(0, ZROWS * (HALF // 16), _zfill, 0)

    for p in range(NP):
        pltpu.sync_copy(g_hbm.at[p * NW + wid], gv)

        def _zacc(k, carry):
            b = s + NS * k

            @pl.when(b < NZB)
            def _():
                pltpu.sync_copy(zb, acc.at[pl.ds(b * ZROWS, ZROWS)])

            return carry

        lax.fori_loop(0, (NZB + NS - 1) // NS, _zacc, 0)

        plsc.subcore_barrier()

        for b in range(NBUF):
            pltpu.async_copy(xw_hbm.at[gv.at[b]], bufs[b], sems[b])

        def _step(jj, carry):
            j = jj * NBUF
            for b in range(NBUF):
                pltpu.make_async_copy(xw_hbm.at[gv.at[j + b]], bufs[b],
                                      sems[b]).wait()
                pltpu.sync_copy(bufs[b], acc.at[dv.at[j + b]], add=True)

                @pl.when(j + b + NBUF < NCH)
                def _():
                    pltpu.async_copy(xw_hbm.at[gv.at[j + b + NBUF]], bufs[b],
                                     sems[b])

            return carry

        lax.fori_loop(0, NCH // NBUF, _step, 0)
        plsc.subcore_barrier()

        def _drain(k, carry):
            b = s + NS * k

            @pl.when(b < NZB)
            def _():
                pltpu.sync_copy(
                    acc.at[pl.ds(b * ZROWS, ZROWS)],
                    out_hbm.at[pl.ds((p * NC + c) * N_NODES + b * ZROWS, ZROWS)],
                )

            return carry

        lax.fori_loop(0, (NZB + NS - 1) // NS, _drain, 0)


@functools.cache
def _sc_gather_scatter():
    return pl.kernel(
        _sc_body,
        out_type=jax.ShapeDtypeStruct((NP * NC * N_NODES, HALF), jnp.float32),
        mesh=plsc.VectorSubcoreMesh(core_axis_name="c", subcore_axis_name="s"),
        scratch_types=[
            pltpu.VMEM((NCH, CHUNK), jnp.int32),
            pltpu.VMEM((NCH, CHUNK), jnp.int32),
            [pltpu.VMEM((CHUNK, HALF), jnp.float32) for _ in range(NBUF)],
            pltpu.VMEM((ZROWS, HALF), jnp.float32),
            pltpu.VMEM_SHARED((ACC_ROWS, HALF), jnp.float32),
            [pltpu.SemaphoreType.DMA for _ in range(NBUF)],
        ],
        compiler_params=pltpu.CompilerParams(use_tc_tiling_on_sc=False),
    )


def _combine_body(pa0, pb0, pa1, pb1, xl_ref, b_ref, o_ref):
    left = pa0[...] + pb0[...]
    right = pa1[...] + pb1[...]
    o_ref[...] = jnp.concatenate([left, right], axis=1) + xl_ref[...] + b_ref[:1]


def _combine(part, xw, bias2d):
    return pl.pallas_call(
        _combine_body,
        grid=(NB,),
        in_specs=[
            pl.BlockSpec((BLK, HALF), lambda nb: (nb, 0)),
            pl.BlockSpec((BLK, HALF), lambda nb: (NB + nb, 0)),
            pl.BlockSpec((BLK, HALF), lambda nb: (2 * NB + nb, 0)),
            pl.BlockSpec((BLK, HALF), lambda nb: (3 * NB + nb, 0)),
            pl.BlockSpec((BLK, OUT_FEAT), lambda nb: (NUM_RELS * NB + nb, 0)),
            pl.BlockSpec((8, OUT_FEAT), lambda nb: (0, 0)),
        ],
        out_specs=pl.BlockSpec((BLK, OUT_FEAT), lambda nb: (nb, 0)),
        out_shape=jax.ShapeDtypeStruct((N_NODES, OUT_FEAT), jnp.float32),
    )(part, part, part, part, xw, bias2d)


def kernel(feat, edge_index, etype, weight, h_bias, loop_weight):
    feat = feat.astype(jnp.float32)
    src = edge_index[0].astype(jnp.int32)
    dst = edge_index[1].astype(jnp.int32)
    et = etype.astype(jnp.int32)

    w_all = jnp.concatenate([weight, loop_weight[None]], axis=0)
    xw = _compute_xw(feat, w_all)
    xw_half = xw.reshape(R17 * N_NODES * NP, HALF)

    g2 = (2 * (et * N_NODES + src)).reshape(NW, EPW)
    g6 = jnp.stack([g2, g2 + 1]).reshape(NP * NW, NCH, CHUNK)
    d3 = dst.reshape(NW, NCH, CHUNK)

    part = _sc_gather_scatter()(xw_half, g6, d3)

    bias2d = jnp.broadcast_to(h_bias.reshape(1, OUT_FEAT), (8, OUT_FEAT))
    return _combine(part, xw, bias2d)


# speedup vs baseline: 1.0180x; 1.0180x over previous
import functools

import jax
import jax.numpy as jnp
from jax import lax
from jax.experimental import pallas as pl
from jax.experimental.pallas import tpu as pltpu
from jax.experimental.pallas import tpu_sc as plsc

N_NODES = 10000
N_EDGES = 320000
IN_FEAT = 128
OUT_FEAT = 128
NUM_RELS = 16
R17 = NUM_RELS + 1
HALF = OUT_FEAT // 2
NP = 2

NC = 2
NS = 16
NW = NC * NS
EPW = N_EDGES // NW
CHUNK = 250
NCH = 40
NBUF = 2
ACC_ROWS = N_NODES + 16
ZROWS = 80
NZB = N_NODES // ZROWS

NB = 10
BLK = N_NODES // NB


def _xw_body(f_ref, w_ref, o_ref):
    o_ref[...] = jnp.dot(f_ref[...], w_ref[0],
                         preferred_element_type=jnp.float32)


def _compute_xw(feat, w_all):
    return pl.pallas_call(
        _xw_body,
        grid=(NB, R17),
        in_specs=[
            pl.BlockSpec((BLK, IN_FEAT), lambda nb, r: (nb, 0)),
            pl.BlockSpec((1, IN_FEAT, OUT_FEAT), lambda nb, r: (r, 0, 0)),
        ],
        out_specs=pl.BlockSpec((BLK, OUT_FEAT), lambda nb, r: (r * NB + nb, 0)),
        out_shape=jax.ShapeDtypeStruct((R17 * N_NODES, OUT_FEAT), jnp.float32),
    )(feat, w_all)


def _sc_body(xw_hbm, g_hbm, d_hbm, out_hbm, gv, dv, bufs, zb, acc, sems):
    c = lax.axis_index("c")
    s = lax.axis_index("s")
    wid = c * NS + s

    pltpu.sync_copy(d_hbm.at[wid], dv)

    zvec = jnp.zeros((16,), jnp.float32)

    def _zfill(i, carry):
        zb[i // (HALF // 16), pl.ds((i % (HALF // 16)) * 16, 16)] = zvec
        return carry

    lax.fori_loop(0, ZROWS * (HALF // 16), _zfill, 0)

    for p in range(NP):
        pltpu.sync_copy(g_hbm.at[p * NW + wid], gv)

        def _zacc(k, carry):
            b = s + NS * k

            @pl.when(b < NZB)
            def _():
                pltpu.sync_copy(zb, acc.at[pl.ds(b * ZROWS, ZROWS)])

            return carry

        lax.fori_loop(0, (NZB + NS - 1) // NS, _zacc, 0)

        plsc.subcore_barrier()

        for b in range(NBUF):
            pltpu.async_copy(xw_hbm.at[gv.at[b]], bufs[b], sems[b])

        def _step(jj, carry):
            j = jj * NBUF
            for b in range(NBUF):
                pltpu.make_async_copy(xw_hbm.at[gv.at[j + b]], bufs[b],
                                      sems[b]).wait()
                pltpu.sync_copy(bufs[b], acc.at[dv.at[j + b]], add=True)

                @pl.when(j + b + NBUF < NCH)
                def _():
                    pltpu.async_copy(xw_hbm.at[gv.at[j + b + NBUF]], bufs[b],
                                     sems[b])

            return carry

        lax.fori_loop(0, NCH // NBUF, _step, 0)
        plsc.subcore_barrier()

        def _drain(k, carry):
            b = s + NS * k

            @pl.when(b < NZB)
            def _():
                pltpu.sync_copy(
                    acc.at[pl.ds(b * ZROWS, ZROWS)],
                    out_hbm.at[pl.ds((p * NC + c) * N_NODES + b * ZROWS, ZROWS)],
                )

            return carry

        lax.fori_loop(0, (NZB + NS - 1) // NS, _drain, 0)


@functools.cache
def _sc_gather_scatter():
    return pl.kernel(
        _sc_body,
        out_type=jax.ShapeDtypeStruct((NP * NC * N_NODES, HALF), jnp.float32),
        mesh=plsc.VectorSubcoreMesh(core_axis_name="c", subcore_axis_name="s"),
        scratch_types=[
            pltpu.VMEM((NCH, CHUNK), jnp.int32),
            pltpu.VMEM((NCH, CHUNK), jnp.int32),
            [pltpu.VMEM((CHUNK, HALF), jnp.float32) for _ in range(NBUF)],
            pltpu.VMEM((ZROWS, HALF), jnp.float32),
            pltpu.VMEM_SHARED((ACC_ROWS, HALF), jnp.float32),
            [pltpu.SemaphoreType.DMA for _ in range(NBUF)],
        ],
        compiler_params=pltpu.CompilerParams(use_tc_tiling_on_sc=False),
    )


def _combine_body(pa0, pb0, pa1, pb1, xl_ref, b_ref, o_ref):
    left = pa0[...] + pb0[...]
    right = pa1[...] + pb1[...]
    o_ref[...] = jnp.concatenate([left, right], axis=1) + xl_ref[...] + b_ref[:1]


def _combine(part, xw, bias2d):
    return pl.pallas_call(
        _combine_body,
        grid=(NB,),
        in_specs=[
            pl.BlockSpec((BLK, HALF), lambda nb: (nb, 0)),
            pl.BlockSpec((BLK, HALF), lambda nb: (NB + nb, 0)),
            pl.BlockSpec((BLK, HALF), lambda nb: (2 * NB + nb, 0)),
            pl.BlockSpec((BLK, HALF), lambda nb: (3 * NB + nb, 0)),
            pl.BlockSpec((BLK, OUT_FEAT), lambda nb: (NUM_RELS * NB + nb, 0)),
            pl.BlockSpec((8, OUT_FEAT), lambda nb: (0, 0)),
        ],
        out_specs=pl.BlockSpec((BLK, OUT_FEAT), lambda nb: (nb, 0)),
        out_shape=jax.ShapeDtypeStruct((N_NODES, OUT_FEAT), jnp.float32),
    )(part, part, part, part, xw, bias2d)


def kernel(feat, edge_index, etype, weight, h_bias, loop_weight):
    feat = feat.astype(jnp.float32)
    src = edge_index[0].astype(jnp.int32)
    dst = edge_index[1].astype(jnp.int32)
    et = etype.astype(jnp.int32)

    w_all = jnp.concatenate([weight, loop_weight[None]], axis=0)
    xw = _compute_xw(feat, w_all)
    xw_half = xw.reshape(R17 * N_NODES * NP, HALF)

    g2 = (2 * (et * N_NODES + src)).reshape(NW, EPW)
    g6 = jnp.stack([g2, g2 + 1]).reshape(NP * NW, NCH, CHUNK)
    d3 = dst.reshape(NW, NCH, CHUNK)

    part = _sc_gather_scatter()(xw_half, g6, d3)

    bias2d = jnp.broadcast_to(h_bias.reshape(1, OUT_FEAT), (8, OUT_FEAT))
    return _combine(part, xw, bias2d)


# speedup vs baseline: 1.0244x; 1.0063x over previous
import functools

import jax
import jax.numpy as jnp
from jax import lax
from jax.experimental import pallas as pl
from jax.experimental.pallas import tpu as pltpu
from jax.experimental.pallas import tpu_sc as plsc

N_NODES = 10000
N_EDGES = 320000
IN_FEAT = 128
OUT_FEAT = 128
NUM_RELS = 16
R17 = NUM_RELS + 1
HALF = OUT_FEAT // 2
NP = 2

NC = 2
NS = 16
NW = NC * NS
EPW = N_EDGES // NW
CHUNK = 200
NCH = 50
NBUF = 2
ACC_ROWS = N_NODES + 16
ZROWS = 80
NZB = N_NODES // ZROWS

NB = 10
BLK = N_NODES // NB


def _xw_body(f_ref, w_ref, o_ref):
    o_ref[...] = jnp.dot(f_ref[...], w_ref[0],
                         preferred_element_type=jnp.float32)


def _compute_xw(feat, w_all):
    return pl.pallas_call(
        _xw_body,
        grid=(NB, R17),
        in_specs=[
            pl.BlockSpec((BLK, IN_FEAT), lambda nb, r: (nb, 0)),
            pl.BlockSpec((1, IN_FEAT, OUT_FEAT), lambda nb, r: (r, 0, 0)),
        ],
        out_specs=pl.BlockSpec((BLK, OUT_FEAT), lambda nb, r: (r * NB + nb, 0)),
        out_shape=jax.ShapeDtypeStruct((R17 * N_NODES, OUT_FEAT), jnp.float32),
    )(feat, w_all)


def _sc_body(xw_hbm, g_hbm, d_hbm, out_hbm, gv, dv, bufs, zb, acc, sems):
    c = lax.axis_index("c")
    s = lax.axis_index("s")
    wid = c * NS + s

    pltpu.sync_copy(d_hbm.at[wid], dv)

    zvec = jnp.zeros((16,), jnp.float32)

    def _zfill(i, carry):
        zb[i // (HALF // 16), pl.ds((i % (HALF // 16)) * 16, 16)] = zvec
        return carry

    lax.fori_loop(0, ZROWS * (HALF // 16), _zfill, 0)

    for p in range(NP):
        pltpu.sync_copy(g_hbm.at[p * NW + wid], gv)

        def _zacc(k, carry):
            b = s + NS * k

            @pl.when(b < NZB)
            def _():
                pltpu.sync_copy(zb, acc.at[pl.ds(b * ZROWS, ZROWS)])

            return carry

        lax.fori_loop(0, (NZB + NS - 1) // NS, _zacc, 0)

        plsc.subcore_barrier()

        for b in range(NBUF):
            pltpu.async_copy(xw_hbm.at[gv.at[b]], bufs[b], sems[b])

        def _step(jj, carry):
            j = jj * NBUF
            for b in range(NBUF):
                pltpu.make_async_copy(xw_hbm.at[gv.at[j + b]], bufs[b],
                                      sems[b]).wait()
                pltpu.sync_copy(bufs[b], acc.at[dv.at[j + b]], add=True)

                @pl.when(j + b + NBUF < NCH)
                def _():
                    pltpu.async_copy(xw_hbm.at[gv.at[j + b + NBUF]], bufs[b],
                                     sems[b])

            return carry

        lax.fori_loop(0, NCH // NBUF, _step, 0)
        plsc.subcore_barrier()

        def _drain(k, carry):
            b = s + NS * k

            @pl.when(b < NZB)
            def _():
                pltpu.sync_copy(
                    acc.at[pl.ds(b * ZROWS, ZROWS)],
                    out_hbm.at[pl.ds((p * NC + c) * N_NODES + b * ZROWS, ZROWS)],
                )

            return carry

        lax.fori_loop(0, (NZB + NS - 1) // NS, _drain, 0)


@functools.cache
def _sc_gather_scatter():
    return pl.kernel(
        _sc_body,
        out_type=jax.ShapeDtypeStruct((NP * NC * N_NODES, HALF), jnp.float32),
        mesh=plsc.VectorSubcoreMesh(core_axis_name="c", subcore_axis_name="s"),
        scratch_types=[
            pltpu.VMEM((NCH, CHUNK), jnp.int32),
            pltpu.VMEM((NCH, CHUNK), jnp.int32),
            [pltpu.VMEM((CHUNK, HALF), jnp.float32) for _ in range(NBUF)],
            pltpu.VMEM((ZROWS, HALF), jnp.float32),
            pltpu.VMEM_SHARED((ACC_ROWS, HALF), jnp.float32),
            [pltpu.SemaphoreType.DMA for _ in range(NBUF)],
        ],
        compiler_params=pltpu.CompilerParams(use_tc_tiling_on_sc=False),
    )


def _combine_body(pa0, pb0, pa1, pb1, xl_ref, b_ref, o_ref):
    left = pa0[...] + pb0[...]
    right = pa1[...] + pb1[...]
    o_ref[...] = jnp.concatenate([left, right], axis=1) + xl_ref[...] + b_ref[:1]


def _combine(part, xw, bias2d):
    return pl.pallas_call(
        _combine_body,
        grid=(NB,),
        in_specs=[
            pl.BlockSpec((BLK, HALF), lambda nb: (nb, 0)),
            pl.BlockSpec((BLK, HALF), lambda nb: (NB + nb, 0)),
            pl.BlockSpec((BLK, HALF), lambda nb: (2 * NB + nb, 0)),
            pl.BlockSpec((BLK, HALF), lambda nb: (3 * NB + nb, 0)),
            pl.BlockSpec((BLK, OUT_FEAT), lambda nb: (NUM_RELS * NB + nb, 0)),
            pl.BlockSpec((8, OUT_FEAT), lambda nb: (0, 0)),
        ],
        out_specs=pl.BlockSpec((BLK, OUT_FEAT), lambda nb: (nb, 0)),
        out_shape=jax.ShapeDtypeStruct((N_NODES, OUT_FEAT), jnp.float32),
    )(part, part, part, part, xw, bias2d)


def kernel(feat, edge_index, etype, weight, h_bias, loop_weight):
    feat = feat.astype(jnp.float32)
    src = edge_index[0].astype(jnp.int32)
    dst = edge_index[1].astype(jnp.int32)
    et = etype.astype(jnp.int32)

    w_all = jnp.concatenate([weight, loop_weight[None]], axis=0)
    xw = _compute_xw(feat, w_all)
    xw_half = xw.reshape(R17 * N_NODES * NP, HALF)

    g2 = (2 * (et * N_NODES + src)).reshape(NW, EPW)
    g6 = jnp.stack([g2, g2 + 1]).reshape(NP * NW, NCH, CHUNK)
    d3 = dst.reshape(NW, NCH, CHUNK)

    part = _sc_gather_scatter()(xw_half, g6, d3)

    bias2d = jnp.broadcast_to(h_bias.reshape(1, OUT_FEAT), (8, OUT_FEAT))
    return _combine(part, xw, bias2d)
